# Initial kernel scaffold; baseline (speedup 1.0000x reference)
import jax
import jax.numpy as jnp
from jax.experimental import pallas as pl


def kernel(ft, adj, W1, W2, b):
    raise NotImplementedError("write your pallas kernel here")


# speedup vs baseline: 1.0430x; 1.0430x over previous
import jax
import jax.numpy as jnp
from jax.experimental import pallas as pl
from jax.experimental.pallas import tpu as pltpu

_BM = 400


def _body(adj_ref, ft_all_ref, ft_row_ref, w1_ref, w2_ref, b_ref, out_ref):
    neigh = jnp.dot(adj_ref[...], ft_all_ref[...],
                    preferred_element_type=jnp.float32)
    acc = jnp.dot(neigh, w1_ref[...], preferred_element_type=jnp.float32)
    acc = acc + jnp.dot(ft_row_ref[...], w2_ref[...],
                        preferred_element_type=jnp.float32)
    acc = acc + b_ref[...]
    out_ref[...] = jnp.maximum(acc, 0.0)


def kernel(ft, adj, W1, W2, b):
    n, in_ch = ft.shape
    out_ch = W1.shape[1]
    bm = _BM
    assert n % bm == 0
    b2 = b.reshape(1, out_ch)
    return pl.pallas_call(
        _body,
        grid=(n // bm,),
        in_specs=[
            pl.BlockSpec((bm, n), lambda i: (i, 0)),
            pl.BlockSpec((n, in_ch), lambda i: (0, 0)),
            pl.BlockSpec((bm, in_ch), lambda i: (i, 0)),
            pl.BlockSpec((in_ch, out_ch), lambda i: (0, 0)),
            pl.BlockSpec((in_ch, out_ch), lambda i: (0, 0)),
            pl.BlockSpec((1, out_ch), lambda i: (0, 0)),
        ],
        out_specs=pl.BlockSpec((bm, out_ch), lambda i: (i, 0)),
        out_shape=jax.ShapeDtypeStruct((n, out_ch), jnp.float32),
        compiler_params=pltpu.CompilerParams(
            dimension_semantics=("arbitrary",)),
    )(adj, ft, ft, W1, W2, b2)
